# baseline (speedup 1.0000x reference)
import functools
import math

import jax
import jax.numpy as jnp
import numpy as np
from jax import lax
from jax.experimental import pallas as pl
from jax.experimental.pallas import tpu as pltpu
from jax.experimental.pallas import tpu_sc as plsc

ALPHA = 0.1
THETA = 0.5
BN_EPS = 1e-5

NC = 2
NS = 16
K = 128


def _deg_kernel(npad, chunks):
    mesh = plsc.VectorSubcoreMesh(core_axis_name="c", subcore_axis_name="s")
    rpt = npad // NS

    def body(cols_hbm, ones_hbm, zeros_hbm, out_hbm, col_v, ones_v, stage_v,
             acc_sh):
        cid = lax.axis_index("c")
        sid = lax.axis_index("s")
        tile = sid * NC + cid
        pltpu.sync_copy(cols_hbm.at[tile], col_v)
        pltpu.sync_copy(ones_hbm, ones_v)
        pltpu.sync_copy(zeros_hbm.at[pl.ds(sid * rpt, rpt)], stage_v)
        pltpu.sync_copy(stage_v, acc_sh.at[pl.ds(sid * rpt, rpt)])
        plsc.subcore_barrier()

        def step(j, carry):
            pltpu.sync_copy(ones_v, acc_sh.at[col_v.at[j]], add=True)
            return carry

        lax.fori_loop(0, chunks, step, 0)
        plsc.subcore_barrier()
        pltpu.sync_copy(acc_sh.at[pl.ds(sid * rpt, rpt)], stage_v)
        off = pl.multiple_of(cid * npad + sid * rpt, 8)
        pltpu.sync_copy(stage_v, out_hbm.at[pl.ds(off, rpt)])

    return pl.kernel(
        body,
        out_type=jax.ShapeDtypeStruct((NC * npad,), jnp.float32),
        mesh=mesh,
        compiler_params=pltpu.CompilerParams(use_tc_tiling_on_sc=False),
        scratch_types=[
            pltpu.VMEM((chunks, K), jnp.int32),
            pltpu.VMEM((K,), jnp.float32),
            pltpu.VMEM((rpt,), jnp.float32),
            pltpu.VMEM_SHARED((npad,), jnp.float32),
        ],
    )


RING = 8


def _row_chunks(total, step):
    out = []
    w = 0
    while w < total:
        out.append((w, min(step, total - w)))
        w += step
    return out


def _spmm_kernel(npad, chunks, hid2):
    mesh = plsc.VectorSubcoreMesh(core_axis_name="c", subcore_axis_name="s")
    rpt = npad // NS
    ngroups = chunks // RING

    def body(g_a, g_b, rows_hbm, cols_hbm, zeros_hbm, out_a, out_b, *rest):
        row_v, col_v = rest[0], rest[1]
        bufs = rest[2:2 + RING]
        acc_sh = rest[2 + RING]
        g_sh = rest[3 + RING]
        gsems = rest[4 + RING:4 + 2 * RING]
        ssems = rest[4 + 2 * RING:]
        cid = lax.axis_index("c")
        sid = lax.axis_index("s")
        tile = sid * NC + cid
        pltpu.sync_copy(rows_hbm.at[tile], row_v)
        pltpu.sync_copy(cols_hbm.at[tile], col_v)

        for g_hbm, out_hbm in ((g_a, out_a), (g_b, out_b)):
            for w, sz in _row_chunks(rpt, K):
                pltpu.sync_copy(zeros_hbm.at[pl.ds(sid * rpt + w, sz)],
                                bufs[0].at[pl.ds(0, sz)])
                pltpu.sync_copy(bufs[0].at[pl.ds(0, sz)],
                                acc_sh.at[pl.ds(sid * rpt + w, sz)])
                pltpu.sync_copy(g_hbm.at[pl.ds(sid * rpt + w, sz)],
                                bufs[1].at[pl.ds(0, sz)])
                pltpu.sync_copy(bufs[1].at[pl.ds(0, sz)],
                                g_sh.at[pl.ds(sid * rpt + w, sz)])
            plsc.subcore_barrier()

            for b in range(RING):
                pltpu.async_copy(g_sh.at[row_v.at[b]], bufs[b], gsems[b])

            def group(gi, carry):
                j0 = gi * RING
                for b in range(RING):
                    j = j0 + b
                    pltpu.make_async_copy(g_sh.at[row_v.at[j]], bufs[b],
                                          gsems[b]).wait()
                    pltpu.async_copy(bufs[b], acc_sh.at[col_v.at[j]],
                                     ssems[b], add=True)
                for b in range(RING):
                    j = j0 + b
                    pltpu.make_async_copy(bufs[b], acc_sh.at[col_v.at[j]],
                                          ssems[b]).wait()
                    jn = j + RING

                    @pl.when(jn < chunks)
                    def _():
                        pltpu.async_copy(g_sh.at[row_v.at[jn]], bufs[b],
                                         gsems[b])
                return carry

            lax.fori_loop(0, ngroups, group, 0)
            plsc.subcore_barrier()
            for w, sz in _row_chunks(rpt, K):
                pltpu.sync_copy(acc_sh.at[pl.ds(sid * rpt + w, sz)],
                                bufs[0].at[pl.ds(0, sz)])
                pltpu.sync_copy(bufs[0].at[pl.ds(0, sz)],
                                out_hbm.at[cid, pl.ds(sid * rpt + w, sz)])

    return pl.kernel(
        body,
        out_type=[jax.ShapeDtypeStruct((NC, npad, hid2), jnp.float32),
                  jax.ShapeDtypeStruct((NC, npad, hid2), jnp.float32)],
        mesh=mesh,
        compiler_params=pltpu.CompilerParams(use_tc_tiling_on_sc=False),
        scratch_types=[
            pltpu.VMEM((chunks, K), jnp.int32),
            pltpu.VMEM((chunks, K), jnp.int32),
        ] + [pltpu.VMEM((K, hid2), jnp.float32) for _ in range(RING)] + [
            pltpu.VMEM_SHARED((npad, hid2), jnp.float32),
            pltpu.VMEM_SHARED((npad, hid2), jnp.float32),
        ] + [pltpu.SemaphoreType.DMA for _ in range(2 * RING)],
    )


def _init_call(x, w0, b0, degp, br, npad):
    n, in_ch = x.shape
    hid = w0.shape[0]
    hid2 = hid // 2

    def body(x_ref, w0_ref, b0_ref, deg_ref, h_ref, ga_ref, gb_ref, dinv_ref):
        h = lax.dot_general(x_ref[...], w0_ref[...], (((1,), (1,)), ((), ())),
                            preferred_element_type=jnp.float32)
        h = jnp.maximum(h + b0_ref[...], 0.0)
        deg = 1.0 + deg_ref[:, 0] + deg_ref[:, 1]
        dinv = lax.rsqrt(deg)[:, None]
        g = dinv * h
        h_ref[...] = h
        ga_ref[...] = g[:, :hid2]
        gb_ref[...] = g[:, hid2:]
        dinv_ref[...] = dinv

    return pl.pallas_call(
        body,
        grid=(n // br,),
        in_specs=[
            pl.BlockSpec((br, in_ch), lambda i: (i, 0)),
            pl.BlockSpec((hid, in_ch), lambda i: (0, 0)),
            pl.BlockSpec((1, hid), lambda i: (0, 0)),
            pl.BlockSpec((br, 2), lambda i: (i, 0)),
        ],
        out_specs=[
            pl.BlockSpec((br, hid), lambda i: (i, 0)),
            pl.BlockSpec((br, hid2), lambda i: (i, 0)),
            pl.BlockSpec((br, hid2), lambda i: (i, 0)),
            pl.BlockSpec((br, 1), lambda i: (i, 0)),
        ],
        out_shape=[
            jax.ShapeDtypeStruct((n, hid), jnp.float32),
            jax.ShapeDtypeStruct((npad, hid2), jnp.float32),
            jax.ShapeDtypeStruct((npad, hid2), jnp.float32),
            jax.ShapeDtypeStruct((n, 1), jnp.float32),
        ],
    )(x, w0, b0.reshape(1, -1), degp)


def _dense_call(sa, sb, ga, gb, x0, dinv, w1l, gam, bet, beta_l, br, npad):
    n, hid = x0.shape
    hid2 = hid // 2
    omb = 1.0 - beta_l
    bn_scale = 1.0 / math.sqrt(1.0 + BN_EPS)

    def body(sa_ref, sb_ref, ga_ref, gb_ref, x0_ref, dinv_ref, w_ref,
             gam_ref, bet_ref, h_ref, g2a_ref, g2b_ref):
        s_a = sa_ref[0] + sa_ref[1] + ga_ref[...]
        s_b = sb_ref[0] + sb_ref[1] + gb_ref[...]
        s = jnp.concatenate([s_a, s_b], axis=1)
        dinv = dinv_ref[...]
        h2 = (1.0 - ALPHA) * (dinv * s) + ALPHA * x0_ref[...]
        mm = lax.dot_general(h2, w_ref[...], (((1,), (0,)), ((), ())),
                             preferred_element_type=jnp.float32)
        h2 = omb * h2 + beta_l * mm
        h2 = gam_ref[...] * (h2 * bn_scale) + bet_ref[...]
        h = jnp.maximum(h2, 0.0)
        g2 = dinv * h
        h_ref[...] = h
        g2a_ref[...] = g2[:, :hid2]
        g2b_ref[...] = g2[:, hid2:]

    return pl.pallas_call(
        body,
        grid=(n // br,),
        in_specs=[
            pl.BlockSpec((2, br, hid2), lambda i: (0, i, 0)),
            pl.BlockSpec((2, br, hid2), lambda i: (0, i, 0)),
            pl.BlockSpec((br, hid2), lambda i: (i, 0)),
            pl.BlockSpec((br, hid2), lambda i: (i, 0)),
            pl.BlockSpec((br, hid), lambda i: (i, 0)),
            pl.BlockSpec((br, 1), lambda i: (i, 0)),
            pl.BlockSpec((hid, hid), lambda i: (0, 0)),
            pl.BlockSpec((1, hid), lambda i: (0, 0)),
            pl.BlockSpec((1, hid), lambda i: (0, 0)),
        ],
        out_specs=[
            pl.BlockSpec((br, hid), lambda i: (i, 0)),
            pl.BlockSpec((br, hid2), lambda i: (i, 0)),
            pl.BlockSpec((br, hid2), lambda i: (i, 0)),
        ],
        out_shape=[
            jax.ShapeDtypeStruct((n, hid), jnp.float32),
            jax.ShapeDtypeStruct((npad, hid2), jnp.float32),
            jax.ShapeDtypeStruct((npad, hid2), jnp.float32),
        ],
    )(sa, sb, ga, gb, x0, dinv, w1l, gam, bet)


def _final_call(h, w_out, b_out, br):
    n, hid = h.shape
    out = w_out.shape[0]

    def body(h_ref, w_ref, b_ref, o_ref):
        o = lax.dot_general(h_ref[...], w_ref[...], (((1,), (1,)), ((), ())),
                            preferred_element_type=jnp.float32)
        o_ref[...] = o + b_ref[...]

    return pl.pallas_call(
        body,
        grid=(n // br,),
        in_specs=[
            pl.BlockSpec((br, hid), lambda i: (i, 0)),
            pl.BlockSpec((out, hid), lambda i: (0, 0)),
            pl.BlockSpec((1, out), lambda i: (0, 0)),
        ],
        out_specs=pl.BlockSpec((br, out), lambda i: (i, 0)),
        out_shape=jax.ShapeDtypeStruct((n, out), jnp.float32),
    )(h, w_out, b_out.reshape(1, -1))


def kernel(x, edge_index, W0, b0, W1, bn_gamma, bn_beta, W_out, b_out):
    n, _ = x.shape
    hid = W0.shape[0]
    num_layers = W1.shape[0]
    e = edge_index.shape[1]

    npad = ((n + 1 + 127) // 128) * 128
    per = NC * NS * K * RING
    e_pad = ((e + per - 1) // per) * per
    chunks = e_pad // (NC * NS * K)
    br = 2000 if n % 2000 == 0 else n

    row = edge_index[0]
    col = edge_index[1]
    pad_e = e_pad - e
    rows_t = jnp.concatenate(
        [row, jnp.zeros((pad_e,), jnp.int32)]).reshape(NC * NS, chunks, K)
    cols_t = jnp.concatenate(
        [col, jnp.full((pad_e,), n, jnp.int32)]).reshape(NC * NS, chunks, K)

    ones_k = jnp.ones((K,), jnp.float32)
    zeros1 = jnp.zeros((npad,), jnp.float32)
    zeros2 = jnp.zeros((npad, hid // 2), jnp.float32)

    degp = _deg_kernel(npad, chunks)(cols_t, ones_k, zeros1)
    h, ga, gb, dinv = _init_call(x, W0, b0, degp.reshape(NC, npad).T, br,
                                 npad)
    x0 = h
    spmm = _spmm_kernel(npad, chunks, hid // 2)
    for l in range(num_layers):
        sa, sb = spmm(ga, gb, rows_t, cols_t, zeros2)
        beta_l = float(np.log(THETA / (l + 1) + 1.0))
        h, ga, gb = _dense_call(sa, sb, ga, gb, x0, dinv, W1[l],
                                bn_gamma[l].reshape(1, -1),
                                bn_beta[l].reshape(1, -1), beta_l, br, npad)
    return _final_call(h, W_out, b_out, br)


# speedup vs baseline: 1.0874x; 1.0874x over previous
import functools
import math

import jax
import jax.numpy as jnp
import numpy as np
from jax import lax
from jax.experimental import pallas as pl
from jax.experimental.pallas import tpu as pltpu
from jax.experimental.pallas import tpu_sc as plsc

ALPHA = 0.1
THETA = 0.5
BN_EPS = 1e-5

NC = 2
NS = 16
K = 128


def _deg_kernel(npad, chunks):
    mesh = plsc.VectorSubcoreMesh(core_axis_name="c", subcore_axis_name="s")
    rpt = npad // NS

    def body(cols_hbm, ones_hbm, zeros_hbm, out_hbm, col_v, ones_v, stage_v,
             acc_sh):
        cid = lax.axis_index("c")
        sid = lax.axis_index("s")
        tile = sid * NC + cid
        pltpu.sync_copy(cols_hbm.at[tile], col_v)
        pltpu.sync_copy(ones_hbm, ones_v)
        pltpu.sync_copy(zeros_hbm.at[pl.ds(sid * rpt, rpt)], stage_v)
        pltpu.sync_copy(stage_v, acc_sh.at[pl.ds(sid * rpt, rpt)])
        plsc.subcore_barrier()

        def step(j, carry):
            pltpu.sync_copy(ones_v, acc_sh.at[col_v.at[j]], add=True)
            return carry

        lax.fori_loop(0, chunks, step, 0)
        plsc.subcore_barrier()
        pltpu.sync_copy(acc_sh.at[pl.ds(sid * rpt, rpt)], stage_v)
        off = pl.multiple_of(cid * npad + sid * rpt, 8)
        pltpu.sync_copy(stage_v, out_hbm.at[pl.ds(off, rpt)])

    return pl.kernel(
        body,
        out_type=jax.ShapeDtypeStruct((NC * npad,), jnp.float32),
        mesh=mesh,
        compiler_params=pltpu.CompilerParams(use_tc_tiling_on_sc=False),
        scratch_types=[
            pltpu.VMEM((chunks, K), jnp.int32),
            pltpu.VMEM((K,), jnp.float32),
            pltpu.VMEM((rpt,), jnp.float32),
            pltpu.VMEM_SHARED((npad,), jnp.float32),
        ],
    )


RING = 8


def _row_chunks(total, step):
    out = []
    w = 0
    while w < total:
        out.append((w, min(step, total - w)))
        w += step
    return out


def _spmm_kernel(npad, chunks, hid2):
    mesh = plsc.VectorSubcoreMesh(core_axis_name="c", subcore_axis_name="s")
    rpt = npad // NS
    ngroups = chunks // RING

    def body(g_a, g_b, rows_hbm, cols_hbm, out_a, out_b, *rest):
        row_v, col_v, zbuf = rest[0], rest[1], rest[2]
        bufs = rest[3:3 + RING]
        acc_sh = rest[3 + RING]
        g_sh = rest[4 + RING]
        gsems = rest[5 + RING:5 + 2 * RING]
        ssems = rest[5 + 2 * RING:]
        cid = lax.axis_index("c")
        sid = lax.axis_index("s")
        tile = sid * NC + cid
        pltpu.sync_copy(rows_hbm.at[tile], row_v)
        pltpu.sync_copy(cols_hbm.at[tile], col_v)

        def zfill(r, carry):
            for c in range(hid2 // 16):
                zbuf[r, pl.ds(c * 16, 16)] = jnp.zeros((16,), jnp.float32)
            return carry

        lax.fori_loop(0, K, zfill, 0)

        for g_hbm, out_hbm in ((g_a, out_a), (g_b, out_b)):
            for w, sz in _row_chunks(rpt, K):
                pltpu.sync_copy(g_hbm.at[pl.ds(sid * rpt + w, sz)],
                                bufs[1].at[pl.ds(0, sz)])
                pltpu.sync_copy(bufs[1].at[pl.ds(0, sz)],
                                g_sh.at[pl.ds(sid * rpt + w, sz)])

                @pl.when(cid == 0)
                def _():
                    pltpu.sync_copy(bufs[1].at[pl.ds(0, sz)],
                                    acc_sh.at[pl.ds(sid * rpt + w, sz)])

                @pl.when(cid == 1)
                def _():
                    pltpu.sync_copy(zbuf.at[pl.ds(0, sz)],
                                    acc_sh.at[pl.ds(sid * rpt + w, sz)])
            plsc.subcore_barrier()

            for b in range(RING):
                pltpu.async_copy(g_sh.at[row_v.at[b]], bufs[b], gsems[b])

            def group(gi, carry):
                j0 = gi * RING
                for b in range(RING):
                    j = j0 + b
                    pltpu.make_async_copy(g_sh.at[row_v.at[j]], bufs[b],
                                          gsems[b]).wait()
                    pltpu.async_copy(bufs[b], acc_sh.at[col_v.at[j]],
                                     ssems[b], add=True)
                for b in range(RING):
                    j = j0 + b
                    pltpu.make_async_copy(bufs[b], acc_sh.at[col_v.at[j]],
                                          ssems[b]).wait()
                    jn = j + RING

                    @pl.when(jn < chunks)
                    def _():
                        pltpu.async_copy(g_sh.at[row_v.at[jn]], bufs[b],
                                         gsems[b])
                return carry

            lax.fori_loop(0, ngroups, group, 0)
            plsc.subcore_barrier()
            for w, sz in _row_chunks(rpt, K):
                pltpu.sync_copy(acc_sh.at[pl.ds(sid * rpt + w, sz)],
                                bufs[0].at[pl.ds(0, sz)])
                pltpu.sync_copy(bufs[0].at[pl.ds(0, sz)],
                                out_hbm.at[cid, pl.ds(sid * rpt + w, sz)])

    return pl.kernel(
        body,
        out_type=[jax.ShapeDtypeStruct((NC, npad, hid2), jnp.float32),
                  jax.ShapeDtypeStruct((NC, npad, hid2), jnp.float32)],
        mesh=mesh,
        compiler_params=pltpu.CompilerParams(use_tc_tiling_on_sc=False),
        scratch_types=[
            pltpu.VMEM((chunks, K), jnp.int32),
            pltpu.VMEM((chunks, K), jnp.int32),
            pltpu.VMEM((K, hid2), jnp.float32),
        ] + [pltpu.VMEM((K, hid2), jnp.float32) for _ in range(RING)] + [
            pltpu.VMEM_SHARED((npad, hid2), jnp.float32),
            pltpu.VMEM_SHARED((npad, hid2), jnp.float32),
        ] + [pltpu.SemaphoreType.DMA for _ in range(2 * RING)],
    )


def _init_call(x, w0, b0, degp, br, npad):
    n, in_ch = x.shape
    hid = w0.shape[0]
    hid2 = hid // 2

    def body(x_ref, w0_ref, b0_ref, deg_ref, h_ref, ga_ref, gb_ref, dinv_ref):
        h = lax.dot_general(x_ref[...], w0_ref[...], (((1,), (1,)), ((), ())),
                            preferred_element_type=jnp.float32)
        h = jnp.maximum(h + b0_ref[...], 0.0)
        deg = 1.0 + deg_ref[:, 0] + deg_ref[:, 1]
        dinv = lax.rsqrt(deg)[:, None]
        g = dinv * h
        h_ref[...] = h
        ga_ref[...] = g[:, :hid2]
        gb_ref[...] = g[:, hid2:]
        dinv_ref[...] = dinv

    return pl.pallas_call(
        body,
        grid=(n // br,),
        in_specs=[
            pl.BlockSpec((br, in_ch), lambda i: (i, 0)),
            pl.BlockSpec((hid, in_ch), lambda i: (0, 0)),
            pl.BlockSpec((1, hid), lambda i: (0, 0)),
            pl.BlockSpec((br, 2), lambda i: (i, 0)),
        ],
        out_specs=[
            pl.BlockSpec((br, hid), lambda i: (i, 0)),
            pl.BlockSpec((br, hid2), lambda i: (i, 0)),
            pl.BlockSpec((br, hid2), lambda i: (i, 0)),
            pl.BlockSpec((br, 1), lambda i: (i, 0)),
        ],
        out_shape=[
            jax.ShapeDtypeStruct((n, hid), jnp.float32),
            jax.ShapeDtypeStruct((npad, hid2), jnp.float32),
            jax.ShapeDtypeStruct((npad, hid2), jnp.float32),
            jax.ShapeDtypeStruct((n, 1), jnp.float32),
        ],
    )(x, w0, b0.reshape(1, -1), degp)


def _layer_h2(sa_ref, sb_ref, x0_ref, dinv_ref, w_ref, gam_ref, bet_ref,
              beta_l):
    omb = 1.0 - beta_l
    bn_scale = 1.0 / math.sqrt(1.0 + BN_EPS)
    s = jnp.concatenate([sa_ref[0] + sa_ref[1], sb_ref[0] + sb_ref[1]],
                        axis=1)
    dinv = dinv_ref[...]
    h2 = (1.0 - ALPHA) * (dinv * s) + ALPHA * x0_ref[...]
    mm = lax.dot_general(h2, w_ref[...], (((1,), (0,)), ((), ())),
                         preferred_element_type=jnp.float32)
    h2 = omb * h2 + beta_l * mm
    h2 = gam_ref[...] * (h2 * bn_scale) + bet_ref[...]
    return jnp.maximum(h2, 0.0), dinv


def _dense_mid_call(sa, sb, x0, dinv, w1l, gam, bet, beta_l, br, npad):
    n, hid = x0.shape
    hid2 = hid // 2

    def body(sa_ref, sb_ref, x0_ref, dinv_ref, w_ref, gam_ref, bet_ref,
             g2a_ref, g2b_ref):
        h, dinv = _layer_h2(sa_ref, sb_ref, x0_ref, dinv_ref, w_ref,
                            gam_ref, bet_ref, beta_l)
        g2 = dinv * h
        g2a_ref[...] = g2[:, :hid2]
        g2b_ref[...] = g2[:, hid2:]

    return pl.pallas_call(
        body,
        grid=(n // br,),
        in_specs=[
            pl.BlockSpec((2, br, hid2), lambda i: (0, i, 0)),
            pl.BlockSpec((2, br, hid2), lambda i: (0, i, 0)),
            pl.BlockSpec((br, hid), lambda i: (i, 0)),
            pl.BlockSpec((br, 1), lambda i: (i, 0)),
            pl.BlockSpec((hid, hid), lambda i: (0, 0)),
            pl.BlockSpec((1, hid), lambda i: (0, 0)),
            pl.BlockSpec((1, hid), lambda i: (0, 0)),
        ],
        out_specs=[
            pl.BlockSpec((br, hid2), lambda i: (i, 0)),
            pl.BlockSpec((br, hid2), lambda i: (i, 0)),
        ],
        out_shape=[
            jax.ShapeDtypeStruct((npad, hid2), jnp.float32),
            jax.ShapeDtypeStruct((npad, hid2), jnp.float32),
        ],
    )(sa, sb, x0, dinv, w1l, gam, bet)


def _dense_final_call(sa, sb, x0, dinv, w1l, gam, bet, w_out, b_out,
                      beta_l, br):
    n, hid = x0.shape
    hid2 = hid // 2
    out = w_out.shape[0]

    def body(sa_ref, sb_ref, x0_ref, dinv_ref, w_ref, gam_ref, bet_ref,
             wo_ref, bo_ref, o_ref):
        h, _ = _layer_h2(sa_ref, sb_ref, x0_ref, dinv_ref, w_ref,
                         gam_ref, bet_ref, beta_l)
        o = lax.dot_general(h, wo_ref[...], (((1,), (1,)), ((), ())),
                            preferred_element_type=jnp.float32)
        o_ref[...] = o + bo_ref[...]

    return pl.pallas_call(
        body,
        grid=(n // br,),
        in_specs=[
            pl.BlockSpec((2, br, hid2), lambda i: (0, i, 0)),
            pl.BlockSpec((2, br, hid2), lambda i: (0, i, 0)),
            pl.BlockSpec((br, hid), lambda i: (i, 0)),
            pl.BlockSpec((br, 1), lambda i: (i, 0)),
            pl.BlockSpec((hid, hid), lambda i: (0, 0)),
            pl.BlockSpec((1, hid), lambda i: (0, 0)),
            pl.BlockSpec((1, hid), lambda i: (0, 0)),
            pl.BlockSpec((out, hid), lambda i: (0, 0)),
            pl.BlockSpec((1, out), lambda i: (0, 0)),
        ],
        out_specs=pl.BlockSpec((br, out), lambda i: (i, 0)),
        out_shape=jax.ShapeDtypeStruct((n, out), jnp.float32),
    )(sa, sb, x0, dinv, w1l, gam, bet, w_out, b_out.reshape(1, -1))


def kernel(x, edge_index, W0, b0, W1, bn_gamma, bn_beta, W_out, b_out):
    n, _ = x.shape
    hid = W0.shape[0]
    num_layers = W1.shape[0]
    e = edge_index.shape[1]

    npad = ((n + 1 + 127) // 128) * 128
    per = NC * NS * K * RING
    e_pad = ((e + per - 1) // per) * per
    chunks = e_pad // (NC * NS * K)
    br = 2000 if n % 2000 == 0 else n

    row = edge_index[0]
    col = edge_index[1]
    pad_e = e_pad - e
    rows_t = jnp.concatenate(
        [row, jnp.zeros((pad_e,), jnp.int32)]).reshape(NC * NS, chunks, K)
    cols_t = jnp.concatenate(
        [col, jnp.full((pad_e,), n, jnp.int32)]).reshape(NC * NS, chunks, K)

    ones_k = jnp.ones((K,), jnp.float32)
    zeros1 = jnp.zeros((npad,), jnp.float32)

    degp = _deg_kernel(npad, chunks)(cols_t, ones_k, zeros1)
    h, ga, gb, dinv = _init_call(x, W0, b0, degp.reshape(NC, npad).T, br,
                                 npad)
    x0 = h
    spmm = _spmm_kernel(npad, chunks, hid // 2)
    for l in range(num_layers):
        sa, sb = spmm(ga, gb, rows_t, cols_t)
        beta_l = float(np.log(THETA / (l + 1) + 1.0))
        gam = bn_gamma[l].reshape(1, -1)
        bet = bn_beta[l].reshape(1, -1)
        if l + 1 < num_layers:
            ga, gb = _dense_mid_call(sa, sb, x0, dinv, W1[l], gam, bet,
                                     beta_l, br, npad)
        else:
            return _dense_final_call(sa, sb, x0, dinv, W1[l], gam, bet,
                                     W_out, b_out, beta_l, br)
